# Initial kernel scaffold; baseline (speedup 1.0000x reference)
import jax
import jax.numpy as jnp
from jax.experimental import pallas as pl


def kernel(residue_index, W):
    raise NotImplementedError("write your pallas kernel here")


# speedup vs baseline: 17.2216x; 17.2216x over previous
import jax
import jax.numpy as jnp
from jax import lax
from jax.experimental import pallas as pl
from jax.experimental.pallas import tpu as pltpu

_NB = 32
_CZ = 128
_R = 8


def _body(r_ref, rcol_ref, wh_ref, wl_ref, o_ref):
    L = r_ref.shape[1]
    r_all = r_ref[0, :]
    ri = rcol_ref[:, 0]
    diff = r_all[None, :] - ri[:, None]
    idx = jnp.clip(diff, -_NB, _NB) + (_NB + 1)
    oh = idx[:, :, None] == lax.broadcasted_iota(jnp.int32, (_R, L, _CZ), 2)
    ohb = oh.astype(jnp.bfloat16).reshape(_R * L, _CZ)
    acc = jnp.dot(ohb, wh_ref[...], preferred_element_type=jnp.float32)
    acc = acc + jnp.dot(ohb, wl_ref[...], preferred_element_type=jnp.float32)
    o_ref[...] = acc.reshape(1, _R, L, _CZ)


def kernel(residue_index, W):
    B, L = residue_index.shape
    V = W.shape[0]
    wh = W.astype(jnp.bfloat16)
    wl = (W - wh.astype(jnp.float32)).astype(jnp.bfloat16)
    wh_p = jnp.zeros((_CZ, _CZ), jnp.bfloat16).at[:V].set(wh)
    wl_p = jnp.zeros((_CZ, _CZ), jnp.bfloat16).at[:V].set(wl)
    out = pl.pallas_call(
        _body,
        grid=(L // _R,),
        in_specs=[
            pl.BlockSpec((1, L), lambda p: (0, 0)),
            pl.BlockSpec((_R, 1), lambda p: (p, 0)),
            pl.BlockSpec((_CZ, _CZ), lambda p: (0, 0)),
            pl.BlockSpec((_CZ, _CZ), lambda p: (0, 0)),
        ],
        out_specs=pl.BlockSpec((1, _R, L, _CZ), lambda p: (0, p, 0, 0)),
        out_shape=jax.ShapeDtypeStruct((B, L, L, _CZ), jnp.float32),
    )(residue_index, residue_index.reshape(L, 1), wh_p, wl_p)
    return out
